# Initial kernel scaffold; baseline (speedup 1.0000x reference)
import functools

import jax
import jax.numpy as jnp
from jax import lax
from jax.experimental import pallas as pl
from jax.experimental.pallas import tpu as pltpu
from jax.experimental.pallas import tpu_sc as plsc

N = 10000
E = 320000
D_IN = 128
D_HID = 16
N_CLS = 40

NC = 2
NS = 16
NW = NC * NS
CH = 128
NPAD = 10240
ROWS = NPAD // NS
K = 80
E_PAD = NW * K * CH
PAD_IDX = NPAD - 1

_MESH = plsc.VectorSubcoreMesh(core_axis_name="c", subcore_axis_name="s")


@functools.partial(
    pl.kernel,
    out_type=jax.ShapeDtypeStruct((NC, NPAD), jnp.float32),
    mesh=_MESH,
    scratch_types=[
        pltpu.VMEM((K, CH), jnp.int32),
        pltpu.VMEM((CH,), jnp.float32),
        pltpu.VMEM_SHARED((NPAD,), jnp.float32),
    ],
)
def _deg_kernel(dst_hbm, zeros_hbm, ones_hbm, out_hbm, idx_v, ones_v, deg_sh):
    cid = lax.axis_index("c")
    sid = lax.axis_index("s")
    g = sid * NC + cid
    pltpu.sync_copy(zeros_hbm.at[pl.ds(sid * ROWS, ROWS)],
                    deg_sh.at[pl.ds(sid * ROWS, ROWS)])
    pltpu.sync_copy(ones_hbm, ones_v)
    pltpu.sync_copy(dst_hbm.at[g], idx_v)
    plsc.subcore_barrier()

    def body(j, carry):
        pltpu.sync_copy(ones_v, deg_sh.at[idx_v.at[j]], add=True)
        return carry

    lax.fori_loop(0, K, body, 0)
    plsc.subcore_barrier()
    pltpu.sync_copy(deg_sh.at[pl.ds(sid * ROWS, ROWS)],
                    out_hbm.at[cid, pl.ds(sid * ROWS, ROWS)])


def _make_agg_kernel(D):
    @functools.partial(
        pl.kernel,
        out_type=jax.ShapeDtypeStruct((NC, NPAD, D), jnp.float32),
        mesh=_MESH,
        scratch_types=[
            pltpu.VMEM((K, CH), jnp.int32),
            pltpu.VMEM((K, CH), jnp.int32),
            pltpu.VMEM((CH, D), jnp.float32),
            pltpu.VMEM((CH, D), jnp.float32),
            pltpu.VMEM_SHARED((NPAD, D), jnp.float32),
            pltpu.SemaphoreType.DMA,
            pltpu.SemaphoreType.DMA,
        ],
    )
    def agg(src_hbm, dst_hbm, hs_hbm, zeros_hbm, out_hbm,
            src_v, dst_v, r0, r1, agg_sh, sem0, sem1):
        cid = lax.axis_index("c")
        sid = lax.axis_index("s")
        g = sid * NC + cid
        pltpu.sync_copy(zeros_hbm.at[pl.ds(sid * ROWS, ROWS)],
                        agg_sh.at[pl.ds(sid * ROWS, ROWS)])
        pltpu.sync_copy(src_hbm.at[g], src_v)
        pltpu.sync_copy(dst_hbm.at[g], dst_v)
        plsc.subcore_barrier()

        def body(i, carry):
            j = 2 * i
            c0 = pltpu.async_copy(hs_hbm.at[src_v.at[j]], r0, sem0)
            c1 = pltpu.async_copy(hs_hbm.at[src_v.at[j + 1]], r1, sem1)
            c0.wait()
            pltpu.sync_copy(r0, agg_sh.at[dst_v.at[j]], add=True)
            c1.wait()
            pltpu.sync_copy(r1, agg_sh.at[dst_v.at[j + 1]], add=True)
            return carry

        lax.fori_loop(0, K // 2, body, 0)
        plsc.subcore_barrier()
        pltpu.sync_copy(agg_sh.at[pl.ds(sid * ROWS, ROWS)],
                        out_hbm.at[cid, pl.ds(sid * ROWS, ROWS)])

    return agg


_agg16 = _make_agg_kernel(D_HID)
_agg40 = _make_agg_kernel(N_CLS)


_GRID_R = 2048
_GRID = NPAD // _GRID_R


def _inv_hs1_body(d0_ref, d1_ref, x_ref, w_ref, inv_ref, hs_ref):
    deg = d0_ref[...] + d1_ref[...] + 1.0
    inv = lax.rsqrt(jnp.maximum(deg, 1.0))
    inv_ref[...] = inv
    hs_ref[...] = jnp.dot(x_ref[...], w_ref[...],
                          preferred_element_type=jnp.float32) * inv


def _inv_hs1(d0, d1, x_pad, w0):
    return pl.pallas_call(
        _inv_hs1_body,
        grid=(_GRID,),
        in_specs=[
            pl.BlockSpec((_GRID_R, 1), lambda i: (i, 0)),
            pl.BlockSpec((_GRID_R, 1), lambda i: (i, 0)),
            pl.BlockSpec((_GRID_R, D_IN), lambda i: (i, 0)),
            pl.BlockSpec((D_IN, D_HID), lambda i: (0, 0)),
        ],
        out_specs=[
            pl.BlockSpec((_GRID_R, 1), lambda i: (i, 0)),
            pl.BlockSpec((_GRID_R, D_HID), lambda i: (i, 0)),
        ],
        out_shape=[
            jax.ShapeDtypeStruct((NPAD, 1), jnp.float32),
            jax.ShapeDtypeStruct((NPAD, D_HID), jnp.float32),
        ],
    )(d0, d1, x_pad, w0)


def _layer1_hs2_body(p0_ref, p1_ref, hs1_ref, inv_ref, b0_ref, w1_ref, hs2_ref):
    agg = p0_ref[...] + p1_ref[...] + hs1_ref[...]
    out1 = jnp.maximum(agg * inv_ref[...] + b0_ref[...], 0.0)
    hs2_ref[...] = jnp.dot(out1, w1_ref[...],
                           preferred_element_type=jnp.float32) * inv_ref[...]


def _layer1_hs2(p0, p1, hs1, inv, b0r, w1):
    return pl.pallas_call(
        _layer1_hs2_body,
        grid=(_GRID,),
        in_specs=[
            pl.BlockSpec((_GRID_R, D_HID), lambda i: (i, 0)),
            pl.BlockSpec((_GRID_R, D_HID), lambda i: (i, 0)),
            pl.BlockSpec((_GRID_R, D_HID), lambda i: (i, 0)),
            pl.BlockSpec((_GRID_R, 1), lambda i: (i, 0)),
            pl.BlockSpec((1, D_HID), lambda i: (0, 0)),
            pl.BlockSpec((D_HID, N_CLS), lambda i: (0, 0)),
        ],
        out_specs=pl.BlockSpec((_GRID_R, N_CLS), lambda i: (i, 0)),
        out_shape=jax.ShapeDtypeStruct((NPAD, N_CLS), jnp.float32),
    )(p0, p1, hs1, inv, b0r, w1)


def _layer2_out_body(p0_ref, p1_ref, hs2_ref, inv_ref, b1_ref, out_ref):
    agg = p0_ref[...] + p1_ref[...] + hs2_ref[...]
    out_ref[...] = agg * inv_ref[...] + b1_ref[...]


def _layer2_out(p0, p1, hs2, inv, b1r):
    return pl.pallas_call(
        _layer2_out_body,
        grid=(_GRID,),
        in_specs=[
            pl.BlockSpec((_GRID_R, N_CLS), lambda i: (i, 0)),
            pl.BlockSpec((_GRID_R, N_CLS), lambda i: (i, 0)),
            pl.BlockSpec((_GRID_R, N_CLS), lambda i: (i, 0)),
            pl.BlockSpec((_GRID_R, 1), lambda i: (i, 0)),
            pl.BlockSpec((1, N_CLS), lambda i: (0, 0)),
        ],
        out_specs=pl.BlockSpec((_GRID_R, N_CLS), lambda i: (i, 0)),
        out_shape=jax.ShapeDtypeStruct((NPAD, N_CLS), jnp.float32),
    )(p0, p1, hs2, inv, b1r)


def kernel(x, edge_index, W0, b0, W1, b1):
    src = edge_index[0].astype(jnp.int32)
    dst = edge_index[1].astype(jnp.int32)
    pad = jnp.full((E_PAD - E,), PAD_IDX, dtype=jnp.int32)
    src_t = jnp.concatenate([src, pad]).reshape(NW, K, CH)
    dst_t = jnp.concatenate([dst, pad]).reshape(NW, K, CH)

    x_pad = jnp.zeros((NPAD, D_IN), jnp.float32).at[:N].set(x)
    z1 = jnp.zeros((NPAD,), jnp.float32)
    z16 = jnp.zeros((NPAD, D_HID), jnp.float32)
    z40 = jnp.zeros((NPAD, N_CLS), jnp.float32)
    ones = jnp.ones((CH,), jnp.float32)

    degp = _deg_kernel(dst_t, z1, ones)
    d0 = degp[0].reshape(NPAD, 1)
    d1 = degp[1].reshape(NPAD, 1)
    inv, hs1 = _inv_hs1(d0, d1, x_pad, W0)

    p1 = _agg16(src_t, dst_t, hs1, z16)
    hs2 = _layer1_hs2(p1[0], p1[1], hs1, inv, b0.reshape(1, D_HID), W1)

    p2 = _agg40(src_t, dst_t, hs2, z40)
    out = _layer2_out(p2[0], p2[1], hs2, inv, b1.reshape(1, N_CLS))
    return out[:N]


# speedup vs baseline: 22.9042x; 22.9042x over previous
import functools

import jax
import jax.numpy as jnp
from jax import lax
from jax.experimental import pallas as pl
from jax.experimental.pallas import tpu as pltpu
from jax.experimental.pallas import tpu_sc as plsc

N = 10000
E = 320000
D_IN = 128
D_HID = 16
N_CLS = 40

NC = 2
NS = 16
NW = NC * NS
CH = 128
NPAD = 10240
ROWS = NPAD // NS
K = 80
E_PAD = NW * K * CH
PAD_IDX = NPAD - 1

_MESH = plsc.VectorSubcoreMesh(core_axis_name="c", subcore_axis_name="s")
_SC_PARAMS = pltpu.CompilerParams(use_tc_tiling_on_sc=False)


@functools.partial(
    pl.kernel,
    out_type=jax.ShapeDtypeStruct((NC, NPAD), jnp.float32),
    mesh=_MESH,
    compiler_params=_SC_PARAMS,
    scratch_types=[
        pltpu.VMEM((K, CH), jnp.int32),
        pltpu.VMEM((CH,), jnp.float32),
        pltpu.VMEM_SHARED((NPAD,), jnp.float32),
    ],
)
def _deg_kernel(dst_hbm, zeros_hbm, ones_hbm, out_hbm, idx_v, ones_v, deg_sh):
    cid = lax.axis_index("c")
    sid = lax.axis_index("s")
    g = sid * NC + cid
    pltpu.sync_copy(zeros_hbm.at[pl.ds(sid * ROWS, ROWS)],
                    deg_sh.at[pl.ds(sid * ROWS, ROWS)])
    pltpu.sync_copy(ones_hbm, ones_v)
    pltpu.sync_copy(dst_hbm.at[g], idx_v)
    plsc.subcore_barrier()

    def body(j, carry):
        pltpu.sync_copy(ones_v, deg_sh.at[idx_v.at[j]], add=True)
        return carry

    lax.fori_loop(0, K, body, 0)
    plsc.subcore_barrier()
    pltpu.sync_copy(deg_sh.at[pl.ds(sid * ROWS, ROWS)],
                    out_hbm.at[cid, pl.ds(sid * ROWS, ROWS)])


def _make_agg_kernel(D):
    @functools.partial(
        pl.kernel,
        out_type=jax.ShapeDtypeStruct((NC, NPAD, D), jnp.float32),
        mesh=_MESH,
        compiler_params=_SC_PARAMS,
        scratch_types=[
            pltpu.VMEM((K, CH), jnp.int32),
            pltpu.VMEM((K, CH), jnp.int32),
            pltpu.VMEM((CH, D), jnp.float32),
            pltpu.VMEM((CH, D), jnp.float32),
            pltpu.VMEM_SHARED((NPAD, D), jnp.float32),
            pltpu.SemaphoreType.DMA,
            pltpu.SemaphoreType.DMA,
        ],
    )
    def agg(src_hbm, dst_hbm, hs_hbm, zeros_hbm, out_hbm,
            src_v, dst_v, r0, r1, agg_sh, sem0, sem1):
        cid = lax.axis_index("c")
        sid = lax.axis_index("s")
        g = sid * NC + cid
        pltpu.sync_copy(zeros_hbm.at[pl.ds(sid * ROWS, ROWS)],
                        agg_sh.at[pl.ds(sid * ROWS, ROWS)])
        pltpu.sync_copy(src_hbm.at[g], src_v)
        pltpu.sync_copy(dst_hbm.at[g], dst_v)
        plsc.subcore_barrier()

        def body(i, carry):
            j = 2 * i
            c0 = pltpu.async_copy(hs_hbm.at[src_v.at[j]], r0, sem0)
            c1 = pltpu.async_copy(hs_hbm.at[src_v.at[j + 1]], r1, sem1)
            c0.wait()
            pltpu.sync_copy(r0, agg_sh.at[dst_v.at[j]], add=True)
            c1.wait()
            pltpu.sync_copy(r1, agg_sh.at[dst_v.at[j + 1]], add=True)
            return carry

        lax.fori_loop(0, K // 2, body, 0)
        plsc.subcore_barrier()
        pltpu.sync_copy(agg_sh.at[pl.ds(sid * ROWS, ROWS)],
                        out_hbm.at[cid, pl.ds(sid * ROWS, ROWS)])

    return agg


_agg16 = _make_agg_kernel(D_HID)
_agg40 = _make_agg_kernel(N_CLS)


_GRID_R = 2048
_GRID = NPAD // _GRID_R


def _inv_hs1_body(d0_ref, d1_ref, x_ref, w_ref, inv_ref, hs_ref):
    deg = d0_ref[...] + d1_ref[...] + 1.0
    inv = lax.rsqrt(jnp.maximum(deg, 1.0))
    inv_ref[...] = inv
    hs_ref[...] = jnp.dot(x_ref[...], w_ref[...],
                          preferred_element_type=jnp.float32) * inv


def _inv_hs1(d0, d1, x_pad, w0):
    return pl.pallas_call(
        _inv_hs1_body,
        grid=(_GRID,),
        in_specs=[
            pl.BlockSpec((_GRID_R, 1), lambda i: (i, 0)),
            pl.BlockSpec((_GRID_R, 1), lambda i: (i, 0)),
            pl.BlockSpec((_GRID_R, D_IN), lambda i: (i, 0)),
            pl.BlockSpec((D_IN, D_HID), lambda i: (0, 0)),
        ],
        out_specs=[
            pl.BlockSpec((_GRID_R, 1), lambda i: (i, 0)),
            pl.BlockSpec((_GRID_R, D_HID), lambda i: (i, 0)),
        ],
        out_shape=[
            jax.ShapeDtypeStruct((NPAD, 1), jnp.float32),
            jax.ShapeDtypeStruct((NPAD, D_HID), jnp.float32),
        ],
    )(d0, d1, x_pad, w0)


def _layer1_hs2_body(p0_ref, p1_ref, hs1_ref, inv_ref, b0_ref, w1_ref, hs2_ref):
    agg = p0_ref[...] + p1_ref[...] + hs1_ref[...]
    out1 = jnp.maximum(agg * inv_ref[...] + b0_ref[...], 0.0)
    hs2_ref[...] = jnp.dot(out1, w1_ref[...],
                           preferred_element_type=jnp.float32) * inv_ref[...]


def _layer1_hs2(p0, p1, hs1, inv, b0r, w1):
    return pl.pallas_call(
        _layer1_hs2_body,
        grid=(_GRID,),
        in_specs=[
            pl.BlockSpec((_GRID_R, D_HID), lambda i: (i, 0)),
            pl.BlockSpec((_GRID_R, D_HID), lambda i: (i, 0)),
            pl.BlockSpec((_GRID_R, D_HID), lambda i: (i, 0)),
            pl.BlockSpec((_GRID_R, 1), lambda i: (i, 0)),
            pl.BlockSpec((1, D_HID), lambda i: (0, 0)),
            pl.BlockSpec((D_HID, N_CLS), lambda i: (0, 0)),
        ],
        out_specs=pl.BlockSpec((_GRID_R, N_CLS), lambda i: (i, 0)),
        out_shape=jax.ShapeDtypeStruct((NPAD, N_CLS), jnp.float32),
    )(p0, p1, hs1, inv, b0r, w1)


def _layer2_out_body(p0_ref, p1_ref, hs2_ref, inv_ref, b1_ref, out_ref):
    agg = p0_ref[...] + p1_ref[...] + hs2_ref[...]
    out_ref[...] = agg * inv_ref[...] + b1_ref[...]


def _layer2_out(p0, p1, hs2, inv, b1r):
    return pl.pallas_call(
        _layer2_out_body,
        grid=(_GRID,),
        in_specs=[
            pl.BlockSpec((_GRID_R, N_CLS), lambda i: (i, 0)),
            pl.BlockSpec((_GRID_R, N_CLS), lambda i: (i, 0)),
            pl.BlockSpec((_GRID_R, N_CLS), lambda i: (i, 0)),
            pl.BlockSpec((_GRID_R, 1), lambda i: (i, 0)),
            pl.BlockSpec((1, N_CLS), lambda i: (0, 0)),
        ],
        out_specs=pl.BlockSpec((_GRID_R, N_CLS), lambda i: (i, 0)),
        out_shape=jax.ShapeDtypeStruct((NPAD, N_CLS), jnp.float32),
    )(p0, p1, hs2, inv, b1r)


def kernel(x, edge_index, W0, b0, W1, b1):
    src = edge_index[0].astype(jnp.int32)
    dst = edge_index[1].astype(jnp.int32)
    pad = jnp.full((E_PAD - E,), PAD_IDX, dtype=jnp.int32)
    src_t = jnp.concatenate([src, pad]).reshape(NW, K, CH)
    dst_t = jnp.concatenate([dst, pad]).reshape(NW, K, CH)

    x_pad = jnp.zeros((NPAD, D_IN), jnp.float32).at[:N].set(x)
    z1 = jnp.zeros((NPAD,), jnp.float32)
    z16 = jnp.zeros((NPAD, D_HID), jnp.float32)
    z40 = jnp.zeros((NPAD, N_CLS), jnp.float32)
    ones = jnp.ones((CH,), jnp.float32)

    degp = _deg_kernel(dst_t, z1, ones)
    d0 = degp[0].reshape(NPAD, 1)
    d1 = degp[1].reshape(NPAD, 1)
    inv, hs1 = _inv_hs1(d0, d1, x_pad, W0)

    p1 = _agg16(src_t, dst_t, hs1, z16)
    hs2 = _layer1_hs2(p1[0], p1[1], hs1, inv, b0.reshape(1, D_HID), W1)

    p2 = _agg40(src_t, dst_t, hs2, z40)
    out = _layer2_out(p2[0], p2[1], hs2, inv, b1.reshape(1, N_CLS))
    return out[:N]


# speedup vs baseline: 24.9964x; 1.0913x over previous
import functools

import jax
import jax.numpy as jnp
from jax import lax
from jax.experimental import pallas as pl
from jax.experimental.pallas import tpu as pltpu
from jax.experimental.pallas import tpu_sc as plsc

N = 10000
E = 320000
D_IN = 128
D_HID = 16
N_CLS = 40

NC = 2
NS = 16
NW = NC * NS
CH = 128
NPAD = 10240
ROWS = NPAD // NS
K = 80
E_PAD = NW * K * CH
PAD_IDX = NPAD - 1

_MESH = plsc.VectorSubcoreMesh(core_axis_name="c", subcore_axis_name="s")
_SC_PARAMS = pltpu.CompilerParams(use_tc_tiling_on_sc=False)


NB = 8


@functools.partial(
    pl.kernel,
    out_type=jax.ShapeDtypeStruct((NC, NPAD), jnp.float32),
    mesh=_MESH,
    compiler_params=_SC_PARAMS,
    scratch_types=[
        pltpu.VMEM((K, CH), jnp.int32),
        pltpu.VMEM((CH,), jnp.float32),
        pltpu.VMEM_SHARED((NPAD,), jnp.float32),
    ] + [pltpu.SemaphoreType.DMA] * NB,
)
def _deg_kernel(dst_hbm, zeros_hbm, ones_hbm, out_hbm, idx_v, ones_v, deg_sh,
                *sems):
    cid = lax.axis_index("c")
    sid = lax.axis_index("s")
    g = sid * NC + cid
    pltpu.sync_copy(zeros_hbm.at[pl.ds(sid * ROWS, ROWS)],
                    deg_sh.at[pl.ds(sid * ROWS, ROWS)])
    pltpu.sync_copy(ones_hbm, ones_v)
    pltpu.sync_copy(dst_hbm.at[g], idx_v)
    plsc.subcore_barrier()

    def body(gi, carry):
        base = gi * NB
        cps = [pltpu.async_copy(ones_v, deg_sh.at[idx_v.at[base + b]],
                                sems[b], add=True)
               for b in range(NB)]
        for cp in cps:
            cp.wait()
        return carry

    lax.fori_loop(0, K // NB, body, 0)
    plsc.subcore_barrier()
    pltpu.sync_copy(deg_sh.at[pl.ds(sid * ROWS, ROWS)],
                    out_hbm.at[cid, pl.ds(sid * ROWS, ROWS)])


def _make_agg_kernel(D):
    @functools.partial(
        pl.kernel,
        out_type=jax.ShapeDtypeStruct((NC, NPAD, D), jnp.float32),
        mesh=_MESH,
        compiler_params=_SC_PARAMS,
        scratch_types=[
            pltpu.VMEM((K, CH), jnp.int32),
            pltpu.VMEM((K, CH), jnp.int32),
            pltpu.VMEM_SHARED((NPAD, D), jnp.float32),
        ] + [pltpu.VMEM((CH, D), jnp.float32)] * NB
          + [pltpu.SemaphoreType.DMA] * (2 * NB),
    )
    def agg(src_hbm, dst_hbm, hs_hbm, zeros_hbm, out_hbm,
            src_v, dst_v, agg_sh, *bufs_sems):
        rb = bufs_sems[:NB]
        gsem = bufs_sems[NB:2 * NB]
        ssem = bufs_sems[2 * NB:]
        cid = lax.axis_index("c")
        sid = lax.axis_index("s")
        g = sid * NC + cid
        pltpu.sync_copy(zeros_hbm.at[pl.ds(sid * ROWS, ROWS)],
                        agg_sh.at[pl.ds(sid * ROWS, ROWS)])
        pltpu.sync_copy(src_hbm.at[g], src_v)
        pltpu.sync_copy(dst_hbm.at[g], dst_v)
        plsc.subcore_barrier()

        def body(gi, carry):
            base = gi * NB
            gcps = [pltpu.async_copy(hs_hbm.at[src_v.at[base + b]],
                                     rb[b], gsem[b])
                    for b in range(NB)]
            scps = []
            for b in range(NB):
                gcps[b].wait()
                scps.append(pltpu.async_copy(
                    rb[b], agg_sh.at[dst_v.at[base + b]], ssem[b], add=True))
            for cp in scps:
                cp.wait()
            return carry

        lax.fori_loop(0, K // NB, body, 0)
        plsc.subcore_barrier()
        pltpu.sync_copy(agg_sh.at[pl.ds(sid * ROWS, ROWS)],
                        out_hbm.at[cid, pl.ds(sid * ROWS, ROWS)])

    return agg


_agg16 = _make_agg_kernel(D_HID)
_agg40 = _make_agg_kernel(N_CLS)


_GRID_R = 2048
_GRID = NPAD // _GRID_R


def _inv_hs1_body(d0_ref, d1_ref, x_ref, w_ref, inv_ref, hs_ref):
    deg = d0_ref[...] + d1_ref[...] + 1.0
    inv = lax.rsqrt(jnp.maximum(deg, 1.0))
    inv_ref[...] = inv
    hs_ref[...] = jnp.dot(x_ref[...], w_ref[...],
                          preferred_element_type=jnp.float32) * inv


def _inv_hs1(d0, d1, x_pad, w0):
    return pl.pallas_call(
        _inv_hs1_body,
        grid=(_GRID,),
        in_specs=[
            pl.BlockSpec((_GRID_R, 1), lambda i: (i, 0)),
            pl.BlockSpec((_GRID_R, 1), lambda i: (i, 0)),
            pl.BlockSpec((_GRID_R, D_IN), lambda i: (i, 0)),
            pl.BlockSpec((D_IN, D_HID), lambda i: (0, 0)),
        ],
        out_specs=[
            pl.BlockSpec((_GRID_R, 1), lambda i: (i, 0)),
            pl.BlockSpec((_GRID_R, D_HID), lambda i: (i, 0)),
        ],
        out_shape=[
            jax.ShapeDtypeStruct((NPAD, 1), jnp.float32),
            jax.ShapeDtypeStruct((NPAD, D_HID), jnp.float32),
        ],
    )(d0, d1, x_pad, w0)


def _layer1_hs2_body(p0_ref, p1_ref, hs1_ref, inv_ref, b0_ref, w1_ref, hs2_ref):
    agg = p0_ref[...] + p1_ref[...] + hs1_ref[...]
    out1 = jnp.maximum(agg * inv_ref[...] + b0_ref[...], 0.0)
    hs2_ref[...] = jnp.dot(out1, w1_ref[...],
                           preferred_element_type=jnp.float32) * inv_ref[...]


def _layer1_hs2(p0, p1, hs1, inv, b0r, w1):
    return pl.pallas_call(
        _layer1_hs2_body,
        grid=(_GRID,),
        in_specs=[
            pl.BlockSpec((_GRID_R, D_HID), lambda i: (i, 0)),
            pl.BlockSpec((_GRID_R, D_HID), lambda i: (i, 0)),
            pl.BlockSpec((_GRID_R, D_HID), lambda i: (i, 0)),
            pl.BlockSpec((_GRID_R, 1), lambda i: (i, 0)),
            pl.BlockSpec((1, D_HID), lambda i: (0, 0)),
            pl.BlockSpec((D_HID, N_CLS), lambda i: (0, 0)),
        ],
        out_specs=pl.BlockSpec((_GRID_R, N_CLS), lambda i: (i, 0)),
        out_shape=jax.ShapeDtypeStruct((NPAD, N_CLS), jnp.float32),
    )(p0, p1, hs1, inv, b0r, w1)


def _layer2_out_body(p0_ref, p1_ref, hs2_ref, inv_ref, b1_ref, out_ref):
    agg = p0_ref[...] + p1_ref[...] + hs2_ref[...]
    out_ref[...] = agg * inv_ref[...] + b1_ref[...]


def _layer2_out(p0, p1, hs2, inv, b1r):
    return pl.pallas_call(
        _layer2_out_body,
        grid=(_GRID,),
        in_specs=[
            pl.BlockSpec((_GRID_R, N_CLS), lambda i: (i, 0)),
            pl.BlockSpec((_GRID_R, N_CLS), lambda i: (i, 0)),
            pl.BlockSpec((_GRID_R, N_CLS), lambda i: (i, 0)),
            pl.BlockSpec((_GRID_R, 1), lambda i: (i, 0)),
            pl.BlockSpec((1, N_CLS), lambda i: (0, 0)),
        ],
        out_specs=pl.BlockSpec((_GRID_R, N_CLS), lambda i: (i, 0)),
        out_shape=jax.ShapeDtypeStruct((NPAD, N_CLS), jnp.float32),
    )(p0, p1, hs2, inv, b1r)


def kernel(x, edge_index, W0, b0, W1, b1):
    src = edge_index[0].astype(jnp.int32)
    dst = edge_index[1].astype(jnp.int32)
    pad = jnp.full((E_PAD - E,), PAD_IDX, dtype=jnp.int32)
    src_t = jnp.concatenate([src, pad]).reshape(NW, K, CH)
    dst_t = jnp.concatenate([dst, pad]).reshape(NW, K, CH)

    x_pad = jnp.zeros((NPAD, D_IN), jnp.float32).at[:N].set(x)
    z1 = jnp.zeros((NPAD,), jnp.float32)
    z16 = jnp.zeros((NPAD, D_HID), jnp.float32)
    z40 = jnp.zeros((NPAD, N_CLS), jnp.float32)
    ones = jnp.ones((CH,), jnp.float32)

    degp = _deg_kernel(dst_t, z1, ones)
    d0 = degp[0].reshape(NPAD, 1)
    d1 = degp[1].reshape(NPAD, 1)
    inv, hs1 = _inv_hs1(d0, d1, x_pad, W0)

    p1 = _agg16(src_t, dst_t, hs1, z16)
    hs2 = _layer1_hs2(p1[0], p1[1], hs1, inv, b0.reshape(1, D_HID), W1)

    p2 = _agg40(src_t, dst_t, hs2, z40)
    out = _layer2_out(p2[0], p2[1], hs2, inv, b1.reshape(1, N_CLS))
    return out[:N]


# speedup vs baseline: 25.0644x; 1.0027x over previous
import functools

import jax
import jax.numpy as jnp
from jax import lax
from jax.experimental import pallas as pl
from jax.experimental.pallas import tpu as pltpu
from jax.experimental.pallas import tpu_sc as plsc

N = 10000
E = 320000
D_IN = 128
D_HID = 16
N_CLS = 40

NC = 2
NS = 16
NW = NC * NS
CH = 128
NPAD = 10240
ROWS = NPAD // NS
K = 80
E_PAD = NW * K * CH
PAD_IDX = NPAD - 1

AGG_K0, AGG_K1 = 40, 120
DEG_K0, DEG_K1 = 64, 96
KMAX = max(AGG_K0, AGG_K1, DEG_K0, DEG_K1)
TOTAL_CH = NS * 2 * K
TOTAL_CH_PAD = TOTAL_CH + KMAX

_MESH = plsc.VectorSubcoreMesh(core_axis_name="c", subcore_axis_name="s")
_SC_PARAMS = pltpu.CompilerParams(use_tc_tiling_on_sc=False)


NB = 8


@functools.partial(
    pl.kernel,
    out_type=jax.ShapeDtypeStruct((NC, NPAD), jnp.float32),
    mesh=_MESH,
    compiler_params=_SC_PARAMS,
    scratch_types=[
        pltpu.VMEM((KMAX, CH), jnp.int32),
        pltpu.VMEM((CH,), jnp.float32),
        pltpu.VMEM_SHARED((NPAD,), jnp.float32),
    ] + [pltpu.SemaphoreType.DMA] * NB,
)
def _deg_kernel(dst_hbm, zeros_hbm, ones_hbm, out_hbm, idx_v, ones_v, deg_sh,
                *sems):
    cid = lax.axis_index("c")
    sid = lax.axis_index("s")
    base_ch = jnp.where(cid == 0, sid * DEG_K0, NS * DEG_K0 + sid * DEG_K1)
    n_grp = jnp.where(cid == 0, DEG_K0 // NB, DEG_K1 // NB)
    pltpu.sync_copy(zeros_hbm.at[pl.ds(sid * ROWS, ROWS)],
                    deg_sh.at[pl.ds(sid * ROWS, ROWS)])
    pltpu.sync_copy(ones_hbm, ones_v)
    pltpu.sync_copy(dst_hbm.at[pl.ds(base_ch, KMAX)], idx_v)
    plsc.subcore_barrier()

    def body(gi, carry):
        base = gi * NB
        cps = [pltpu.async_copy(ones_v, deg_sh.at[idx_v.at[base + b]],
                                sems[b], add=True)
               for b in range(NB)]
        for cp in cps:
            cp.wait()
        return carry

    lax.fori_loop(0, n_grp, body, 0)
    plsc.subcore_barrier()
    pltpu.sync_copy(deg_sh.at[pl.ds(sid * ROWS, ROWS)],
                    out_hbm.at[cid, pl.ds(sid * ROWS, ROWS)])


def _make_agg_kernel(D):
    @functools.partial(
        pl.kernel,
        out_type=jax.ShapeDtypeStruct((NC, NPAD, D), jnp.float32),
        mesh=_MESH,
        compiler_params=_SC_PARAMS,
        scratch_types=[
            pltpu.VMEM((KMAX, CH), jnp.int32),
            pltpu.VMEM((KMAX, CH), jnp.int32),
            pltpu.VMEM_SHARED((NPAD, D), jnp.float32),
        ] + [pltpu.VMEM((CH, D), jnp.float32)] * NB
          + [pltpu.SemaphoreType.DMA] * (2 * NB),
    )
    def agg(src_hbm, dst_hbm, hs_hbm, zeros_hbm, out_hbm,
            src_v, dst_v, agg_sh, *bufs_sems):
        rb = bufs_sems[:NB]
        gsem = bufs_sems[NB:2 * NB]
        ssem = bufs_sems[2 * NB:]
        cid = lax.axis_index("c")
        sid = lax.axis_index("s")
        base_ch = jnp.where(cid == 0, sid * AGG_K0, NS * AGG_K0 + sid * AGG_K1)
        n_grp = jnp.where(cid == 0, AGG_K0 // NB, AGG_K1 // NB)
        pltpu.sync_copy(zeros_hbm.at[pl.ds(sid * ROWS, ROWS)],
                        agg_sh.at[pl.ds(sid * ROWS, ROWS)])
        pltpu.sync_copy(src_hbm.at[pl.ds(base_ch, KMAX)], src_v)
        pltpu.sync_copy(dst_hbm.at[pl.ds(base_ch, KMAX)], dst_v)
        plsc.subcore_barrier()

        def body(gi, carry):
            base = gi * NB
            gcps = [pltpu.async_copy(hs_hbm.at[src_v.at[base + b]],
                                     rb[b], gsem[b])
                    for b in range(NB)]
            scps = []
            for b in range(NB):
                gcps[b].wait()
                scps.append(pltpu.async_copy(
                    rb[b], agg_sh.at[dst_v.at[base + b]], ssem[b], add=True))
            for cp in scps:
                cp.wait()
            return carry

        lax.fori_loop(0, n_grp, body, 0)
        plsc.subcore_barrier()
        pltpu.sync_copy(agg_sh.at[pl.ds(sid * ROWS, ROWS)],
                        out_hbm.at[cid, pl.ds(sid * ROWS, ROWS)])

    return agg


_agg16 = _make_agg_kernel(D_HID)
_agg40 = _make_agg_kernel(N_CLS)


_GRID_R = 2048
_GRID = NPAD // _GRID_R


def _inv_hs1_body(d0_ref, d1_ref, x_ref, w_ref, inv_ref, hs_ref):
    deg = d0_ref[...] + d1_ref[...] + 1.0
    inv = lax.rsqrt(jnp.maximum(deg, 1.0))
    inv_ref[...] = inv
    hs_ref[...] = jnp.dot(x_ref[...], w_ref[...],
                          preferred_element_type=jnp.float32) * inv


def _inv_hs1(d0, d1, x_pad, w0):
    return pl.pallas_call(
        _inv_hs1_body,
        grid=(_GRID,),
        in_specs=[
            pl.BlockSpec((_GRID_R, 1), lambda i: (i, 0)),
            pl.BlockSpec((_GRID_R, 1), lambda i: (i, 0)),
            pl.BlockSpec((_GRID_R, D_IN), lambda i: (i, 0)),
            pl.BlockSpec((D_IN, D_HID), lambda i: (0, 0)),
        ],
        out_specs=[
            pl.BlockSpec((_GRID_R, 1), lambda i: (i, 0)),
            pl.BlockSpec((_GRID_R, D_HID), lambda i: (i, 0)),
        ],
        out_shape=[
            jax.ShapeDtypeStruct((NPAD, 1), jnp.float32),
            jax.ShapeDtypeStruct((NPAD, D_HID), jnp.float32),
        ],
    )(d0, d1, x_pad, w0)


def _layer1_hs2_body(p0_ref, p1_ref, hs1_ref, inv_ref, b0_ref, w1_ref, hs2_ref):
    agg = p0_ref[...] + p1_ref[...] + hs1_ref[...]
    out1 = jnp.maximum(agg * inv_ref[...] + b0_ref[...], 0.0)
    hs2_ref[...] = jnp.dot(out1, w1_ref[...],
                           preferred_element_type=jnp.float32) * inv_ref[...]


def _layer1_hs2(p0, p1, hs1, inv, b0r, w1):
    return pl.pallas_call(
        _layer1_hs2_body,
        grid=(_GRID,),
        in_specs=[
            pl.BlockSpec((_GRID_R, D_HID), lambda i: (i, 0)),
            pl.BlockSpec((_GRID_R, D_HID), lambda i: (i, 0)),
            pl.BlockSpec((_GRID_R, D_HID), lambda i: (i, 0)),
            pl.BlockSpec((_GRID_R, 1), lambda i: (i, 0)),
            pl.BlockSpec((1, D_HID), lambda i: (0, 0)),
            pl.BlockSpec((D_HID, N_CLS), lambda i: (0, 0)),
        ],
        out_specs=pl.BlockSpec((_GRID_R, N_CLS), lambda i: (i, 0)),
        out_shape=jax.ShapeDtypeStruct((NPAD, N_CLS), jnp.float32),
    )(p0, p1, hs1, inv, b0r, w1)


def _layer2_out_body(p0_ref, p1_ref, hs2_ref, inv_ref, b1_ref, out_ref):
    agg = p0_ref[...] + p1_ref[...] + hs2_ref[...]
    out_ref[...] = agg * inv_ref[...] + b1_ref[...]


def _layer2_out(p0, p1, hs2, inv, b1r):
    return pl.pallas_call(
        _layer2_out_body,
        grid=(_GRID,),
        in_specs=[
            pl.BlockSpec((_GRID_R, N_CLS), lambda i: (i, 0)),
            pl.BlockSpec((_GRID_R, N_CLS), lambda i: (i, 0)),
            pl.BlockSpec((_GRID_R, N_CLS), lambda i: (i, 0)),
            pl.BlockSpec((_GRID_R, 1), lambda i: (i, 0)),
            pl.BlockSpec((1, N_CLS), lambda i: (0, 0)),
        ],
        out_specs=pl.BlockSpec((_GRID_R, N_CLS), lambda i: (i, 0)),
        out_shape=jax.ShapeDtypeStruct((NPAD, N_CLS), jnp.float32),
    )(p0, p1, hs2, inv, b1r)


def kernel(x, edge_index, W0, b0, W1, b1):
    src = edge_index[0].astype(jnp.int32)
    dst = edge_index[1].astype(jnp.int32)
    pad = jnp.full((TOTAL_CH_PAD * CH - E,), PAD_IDX, dtype=jnp.int32)
    src_t = jnp.concatenate([src, pad]).reshape(TOTAL_CH_PAD, CH)
    dst_t = jnp.concatenate([dst, pad]).reshape(TOTAL_CH_PAD, CH)

    x_pad = jnp.zeros((NPAD, D_IN), jnp.float32).at[:N].set(x)
    z1 = jnp.zeros((NPAD,), jnp.float32)
    z16 = jnp.zeros((NPAD, D_HID), jnp.float32)
    z40 = jnp.zeros((NPAD, N_CLS), jnp.float32)
    ones = jnp.ones((CH,), jnp.float32)

    degp = _deg_kernel(dst_t, z1, ones)
    d0 = degp[0].reshape(NPAD, 1)
    d1 = degp[1].reshape(NPAD, 1)
    inv, hs1 = _inv_hs1(d0, d1, x_pad, W0)

    p1 = _agg16(src_t, dst_t, hs1, z16)
    hs2 = _layer1_hs2(p1[0], p1[1], hs1, inv, b0.reshape(1, D_HID), W1)

    p2 = _agg40(src_t, dst_t, hs2, z40)
    out = _layer2_out(p2[0], p2[1], hs2, inv, b1.reshape(1, N_CLS))
    return out[:N]


# speedup vs baseline: 25.3456x; 1.0112x over previous
import functools

import jax
import jax.numpy as jnp
from jax import lax
from jax.experimental import pallas as pl
from jax.experimental.pallas import tpu as pltpu
from jax.experimental.pallas import tpu_sc as plsc

N = 10000
E = 320000
D_IN = 128
D_HID = 16
N_CLS = 40

NC = 2
NS = 16
NW = NC * NS
CH = 128
NPAD = 10240
ROWS = NPAD // NS
K = 80
E_PAD = NW * K * CH
PAD_IDX = NPAD - 1

AGG_K0, AGG_K1 = 80, 80
DEG_K0, DEG_K1 = 80, 80
KMAX = max(AGG_K0, AGG_K1, DEG_K0, DEG_K1)
TOTAL_CH = NS * 2 * K
TOTAL_CH_PAD = TOTAL_CH + KMAX

_MESH = plsc.VectorSubcoreMesh(core_axis_name="c", subcore_axis_name="s")
_SC_PARAMS = pltpu.CompilerParams(use_tc_tiling_on_sc=False)


NB = 8


@functools.partial(
    pl.kernel,
    out_type=jax.ShapeDtypeStruct((NC, NPAD), jnp.float32),
    mesh=_MESH,
    compiler_params=_SC_PARAMS,
    scratch_types=[
        pltpu.VMEM((KMAX, CH), jnp.int32),
        pltpu.VMEM((CH,), jnp.float32),
        pltpu.VMEM_SHARED((NPAD,), jnp.float32),
    ] + [pltpu.SemaphoreType.DMA] * NB,
)
def _deg_kernel(dst_hbm, zeros_hbm, ones_hbm, out_hbm, idx_v, ones_v, deg_sh,
                *sems):
    cid = lax.axis_index("c")
    sid = lax.axis_index("s")
    base_ch = jnp.where(cid == 0, sid * DEG_K0, NS * DEG_K0 + sid * DEG_K1)
    n_grp = jnp.where(cid == 0, DEG_K0 // NB, DEG_K1 // NB)
    pltpu.sync_copy(zeros_hbm.at[pl.ds(sid * ROWS, ROWS)],
                    deg_sh.at[pl.ds(sid * ROWS, ROWS)])
    pltpu.sync_copy(ones_hbm, ones_v)
    pltpu.sync_copy(dst_hbm.at[pl.ds(base_ch, KMAX)], idx_v)
    plsc.subcore_barrier()

    def body(gi, carry):
        base = gi * NB
        cps = [pltpu.async_copy(ones_v, deg_sh.at[idx_v.at[base + b]],
                                sems[b], add=True)
               for b in range(NB)]
        for cp in cps:
            cp.wait()
        return carry

    lax.fori_loop(0, n_grp, body, 0)
    plsc.subcore_barrier()
    pltpu.sync_copy(deg_sh.at[pl.ds(sid * ROWS, ROWS)],
                    out_hbm.at[cid, pl.ds(sid * ROWS, ROWS)])


def _make_agg_kernel(D):
    @functools.partial(
        pl.kernel,
        out_type=jax.ShapeDtypeStruct((NC, NPAD, D), jnp.float32),
        mesh=_MESH,
        compiler_params=_SC_PARAMS,
        scratch_types=[
            pltpu.VMEM((KMAX, CH), jnp.int32),
            pltpu.VMEM((KMAX, CH), jnp.int32),
            pltpu.VMEM_SHARED((NPAD, D), jnp.float32),
        ] + [pltpu.VMEM((CH, D), jnp.float32)] * NB
          + [pltpu.SemaphoreType.DMA] * (2 * NB),
    )
    def agg(src_hbm, dst_hbm, hs_hbm, zeros_hbm, out_hbm,
            src_v, dst_v, agg_sh, *bufs_sems):
        rb = bufs_sems[:NB]
        gsem = bufs_sems[NB:2 * NB]
        ssem = bufs_sems[2 * NB:]
        cid = lax.axis_index("c")
        sid = lax.axis_index("s")
        base_ch = jnp.where(cid == 0, sid * AGG_K0, NS * AGG_K0 + sid * AGG_K1)
        n_grp = jnp.where(cid == 0, AGG_K0 // NB, AGG_K1 // NB)
        pltpu.sync_copy(zeros_hbm.at[pl.ds(sid * ROWS, ROWS)],
                        agg_sh.at[pl.ds(sid * ROWS, ROWS)])
        pltpu.sync_copy(src_hbm.at[pl.ds(base_ch, KMAX)], src_v)
        pltpu.sync_copy(dst_hbm.at[pl.ds(base_ch, KMAX)], dst_v)
        plsc.subcore_barrier()

        def body(gi, carry):
            base = gi * NB
            gcps = [pltpu.async_copy(hs_hbm.at[src_v.at[base + b]],
                                     rb[b], gsem[b])
                    for b in range(NB)]
            scps = []
            for b in range(NB):
                gcps[b].wait()
                scps.append(pltpu.async_copy(
                    rb[b], agg_sh.at[dst_v.at[base + b]], ssem[b], add=True))
            for cp in scps:
                cp.wait()
            return carry

        lax.fori_loop(0, n_grp, body, 0)
        plsc.subcore_barrier()
        pltpu.sync_copy(agg_sh.at[pl.ds(sid * ROWS, ROWS)],
                        out_hbm.at[cid, pl.ds(sid * ROWS, ROWS)])

    return agg


_agg16 = _make_agg_kernel(D_HID)
_agg40 = _make_agg_kernel(N_CLS)


_GRID_R = 2000
_GRID = N // _GRID_R


def _inv_hs1_body(d0_ref, d1_ref, x_ref, w_ref, inv_ref, hs_ref):
    deg = d0_ref[...] + d1_ref[...] + 1.0
    inv = lax.rsqrt(jnp.maximum(deg, 1.0))
    inv_ref[...] = inv
    hs_ref[...] = jnp.dot(x_ref[...], w_ref[...],
                          preferred_element_type=jnp.float32) * inv


def _inv_hs1(d0, d1, x, w0):
    return pl.pallas_call(
        _inv_hs1_body,
        grid=(_GRID,),
        in_specs=[
            pl.BlockSpec((_GRID_R, 1), lambda i: (i, 0)),
            pl.BlockSpec((_GRID_R, 1), lambda i: (i, 0)),
            pl.BlockSpec((_GRID_R, D_IN), lambda i: (i, 0)),
            pl.BlockSpec((D_IN, D_HID), lambda i: (0, 0)),
        ],
        out_specs=[
            pl.BlockSpec((_GRID_R, 1), lambda i: (i, 0)),
            pl.BlockSpec((_GRID_R, D_HID), lambda i: (i, 0)),
        ],
        out_shape=[
            jax.ShapeDtypeStruct((N, 1), jnp.float32),
            jax.ShapeDtypeStruct((N, D_HID), jnp.float32),
        ],
    )(d0, d1, x, w0)


def _layer1_hs2_body(p0_ref, p1_ref, hs1_ref, inv_ref, b0_ref, w1_ref, hs2_ref):
    agg = p0_ref[...] + p1_ref[...] + hs1_ref[...]
    out1 = jnp.maximum(agg * inv_ref[...] + b0_ref[...], 0.0)
    hs2_ref[...] = jnp.dot(out1, w1_ref[...],
                           preferred_element_type=jnp.float32) * inv_ref[...]


def _layer1_hs2(p0, p1, hs1, inv, b0r, w1):
    return pl.pallas_call(
        _layer1_hs2_body,
        grid=(_GRID,),
        in_specs=[
            pl.BlockSpec((_GRID_R, D_HID), lambda i: (i, 0)),
            pl.BlockSpec((_GRID_R, D_HID), lambda i: (i, 0)),
            pl.BlockSpec((_GRID_R, D_HID), lambda i: (i, 0)),
            pl.BlockSpec((_GRID_R, 1), lambda i: (i, 0)),
            pl.BlockSpec((1, D_HID), lambda i: (0, 0)),
            pl.BlockSpec((D_HID, N_CLS), lambda i: (0, 0)),
        ],
        out_specs=pl.BlockSpec((_GRID_R, N_CLS), lambda i: (i, 0)),
        out_shape=jax.ShapeDtypeStruct((N, N_CLS), jnp.float32),
    )(p0, p1, hs1, inv, b0r, w1)


def _layer2_out_body(p0_ref, p1_ref, hs2_ref, inv_ref, b1_ref, out_ref):
    agg = p0_ref[...] + p1_ref[...] + hs2_ref[...]
    out_ref[...] = agg * inv_ref[...] + b1_ref[...]


def _layer2_out(p0, p1, hs2, inv, b1r):
    return pl.pallas_call(
        _layer2_out_body,
        grid=(_GRID,),
        in_specs=[
            pl.BlockSpec((_GRID_R, N_CLS), lambda i: (i, 0)),
            pl.BlockSpec((_GRID_R, N_CLS), lambda i: (i, 0)),
            pl.BlockSpec((_GRID_R, N_CLS), lambda i: (i, 0)),
            pl.BlockSpec((_GRID_R, 1), lambda i: (i, 0)),
            pl.BlockSpec((1, N_CLS), lambda i: (0, 0)),
        ],
        out_specs=pl.BlockSpec((_GRID_R, N_CLS), lambda i: (i, 0)),
        out_shape=jax.ShapeDtypeStruct((N, N_CLS), jnp.float32),
    )(p0, p1, hs2, inv, b1r)


def kernel(x, edge_index, W0, b0, W1, b1):
    src = edge_index[0].astype(jnp.int32)
    dst = edge_index[1].astype(jnp.int32)
    pad_src = jnp.zeros((TOTAL_CH_PAD * CH - E,), dtype=jnp.int32)
    pad_dst = jnp.full((TOTAL_CH_PAD * CH - E,), PAD_IDX, dtype=jnp.int32)
    src_t = jnp.concatenate([src, pad_src]).reshape(TOTAL_CH_PAD, CH)
    dst_t = jnp.concatenate([dst, pad_dst]).reshape(TOTAL_CH_PAD, CH)

    z1 = jnp.zeros((NPAD,), jnp.float32)
    z16 = jnp.zeros((NPAD, D_HID), jnp.float32)
    z40 = jnp.zeros((NPAD, N_CLS), jnp.float32)
    ones = jnp.ones((CH,), jnp.float32)

    degp = _deg_kernel(dst_t, z1, ones)
    d0 = degp[0].reshape(NPAD, 1)
    d1 = degp[1].reshape(NPAD, 1)
    inv, hs1 = _inv_hs1(d0, d1, x, W0)

    p1 = _agg16(src_t, dst_t, hs1, z16)
    hs2 = _layer1_hs2(p1[0], p1[1], hs1, inv, b0.reshape(1, D_HID), W1)

    p2 = _agg40(src_t, dst_t, hs2, z40)
    return _layer2_out(p2[0], p2[1], hs2, inv, b1.reshape(1, N_CLS))


# speedup vs baseline: 43.1287x; 1.7016x over previous
import functools

import jax
import jax.numpy as jnp
from jax import lax
from jax.experimental import pallas as pl
from jax.experimental.pallas import tpu as pltpu
from jax.experimental.pallas import tpu_sc as plsc

N = 10000
E = 320000
D_IN = 128
D_HID = 16
N_CLS = 40

NC = 2
NS = 16
NW = NC * NS
CH = 128
NPAD = 10240
ROWS = NPAD // NS
K = 80
E_PAD = NW * K * CH
PAD_IDX = NPAD - 1

AGG_K0, AGG_K1 = 80, 80
DEG_K0, DEG_K1 = 80, 80
KMAX = max(AGG_K0, AGG_K1, DEG_K0, DEG_K1)
TOTAL_CH = NS * 2 * K
TOTAL_CH_PAD = TOTAL_CH + KMAX

_MESH = plsc.VectorSubcoreMesh(core_axis_name="c", subcore_axis_name="s")
_SC_PARAMS = pltpu.CompilerParams(use_tc_tiling_on_sc=False)


NB = 8


@functools.partial(
    pl.kernel,
    out_type=jax.ShapeDtypeStruct((NC, NPAD), jnp.float32),
    mesh=_MESH,
    compiler_params=_SC_PARAMS,
    scratch_types=[
        pltpu.VMEM((KMAX, CH), jnp.int32),
        pltpu.VMEM((CH,), jnp.float32),
        pltpu.VMEM_SHARED((NPAD,), jnp.float32),
    ] + [pltpu.SemaphoreType.DMA] * NB,
)
def _deg_kernel(dst_hbm, zeros_hbm, ones_hbm, out_hbm, idx_v, ones_v, deg_sh,
                *sems):
    cid = lax.axis_index("c")
    sid = lax.axis_index("s")
    base_ch = jnp.where(cid == 0, sid * DEG_K0, NS * DEG_K0 + sid * DEG_K1)
    n_grp = jnp.where(cid == 0, DEG_K0 // NB, DEG_K1 // NB)
    pltpu.sync_copy(zeros_hbm.at[pl.ds(sid * ROWS, ROWS)],
                    deg_sh.at[pl.ds(sid * ROWS, ROWS)])
    pltpu.sync_copy(ones_hbm, ones_v)
    pltpu.sync_copy(dst_hbm.at[pl.ds(base_ch, KMAX)], idx_v)
    plsc.subcore_barrier()

    def body(gi, carry):
        base = gi * NB
        cps = [pltpu.async_copy(ones_v, deg_sh.at[idx_v.at[base + b]],
                                sems[b], add=True)
               for b in range(NB)]
        for cp in cps:
            cp.wait()
        return carry

    lax.fori_loop(0, n_grp, body, 0)
    plsc.subcore_barrier()
    pltpu.sync_copy(deg_sh.at[pl.ds(sid * ROWS, ROWS)],
                    out_hbm.at[cid, pl.ds(sid * ROWS, ROWS)])


def _make_agg_kernel(D):
    @functools.partial(
        pl.kernel,
        out_type=jax.ShapeDtypeStruct((NC, NPAD, D), jnp.float32),
        mesh=_MESH,
        compiler_params=_SC_PARAMS,
        scratch_types=[
            pltpu.VMEM((KMAX, CH), jnp.int32),
            pltpu.VMEM((KMAX, CH), jnp.int32),
            pltpu.VMEM_SHARED((NPAD, D), jnp.float32),
            pltpu.VMEM_SHARED((N, D), jnp.float32),
        ] + [pltpu.VMEM((CH, D), jnp.float32)] * NB
          + [pltpu.SemaphoreType.DMA] * (2 * NB),
    )
    def agg(src_hbm, dst_hbm, hs_hbm, zeros_hbm, out_hbm,
            src_v, dst_v, agg_sh, hs_sh, *bufs_sems):
        rb = bufs_sems[:NB]
        gsem = bufs_sems[NB:2 * NB]
        ssem = bufs_sems[2 * NB:]
        cid = lax.axis_index("c")
        sid = lax.axis_index("s")
        base_ch = jnp.where(cid == 0, sid * AGG_K0, NS * AGG_K0 + sid * AGG_K1)
        n_grp = jnp.where(cid == 0, AGG_K0 // NB, AGG_K1 // NB)
        pltpu.sync_copy(zeros_hbm.at[pl.ds(sid * ROWS, ROWS)],
                        agg_sh.at[pl.ds(sid * ROWS, ROWS)])
        pltpu.sync_copy(src_hbm.at[pl.ds(base_ch, KMAX)], src_v)
        pltpu.sync_copy(dst_hbm.at[pl.ds(base_ch, KMAX)], dst_v)
        pltpu.sync_copy(hs_hbm.at[pl.ds(sid * (N // NS), N // NS)],
                        hs_sh.at[pl.ds(sid * (N // NS), N // NS)])
        plsc.subcore_barrier()

        def body(gi, carry):
            base = gi * NB
            gcps = [pltpu.async_copy(hs_sh.at[src_v.at[base + b]],
                                     rb[b], gsem[b])
                    for b in range(NB)]
            scps = []
            for b in range(NB):
                gcps[b].wait()
                scps.append(pltpu.async_copy(
                    rb[b], agg_sh.at[dst_v.at[base + b]], ssem[b], add=True))
            for cp in scps:
                cp.wait()
            return carry

        lax.fori_loop(0, n_grp, body, 0)
        plsc.subcore_barrier()
        pltpu.sync_copy(agg_sh.at[pl.ds(sid * ROWS, ROWS)],
                        out_hbm.at[cid, pl.ds(sid * ROWS, ROWS)])

    return agg


_agg16 = _make_agg_kernel(D_HID)
_agg40 = _make_agg_kernel(N_CLS)


_GRID_R = 2000
_GRID = N // _GRID_R


def _inv_hs1_body(d0_ref, d1_ref, x_ref, w_ref, inv_ref, hs_ref):
    deg = d0_ref[...] + d1_ref[...] + 1.0
    inv = lax.rsqrt(jnp.maximum(deg, 1.0))
    inv_ref[...] = inv
    hs_ref[...] = jnp.dot(x_ref[...], w_ref[...],
                          preferred_element_type=jnp.float32) * inv


def _inv_hs1(d0, d1, x, w0):
    return pl.pallas_call(
        _inv_hs1_body,
        grid=(_GRID,),
        in_specs=[
            pl.BlockSpec((_GRID_R, 1), lambda i: (i, 0)),
            pl.BlockSpec((_GRID_R, 1), lambda i: (i, 0)),
            pl.BlockSpec((_GRID_R, D_IN), lambda i: (i, 0)),
            pl.BlockSpec((D_IN, D_HID), lambda i: (0, 0)),
        ],
        out_specs=[
            pl.BlockSpec((_GRID_R, 1), lambda i: (i, 0)),
            pl.BlockSpec((_GRID_R, D_HID), lambda i: (i, 0)),
        ],
        out_shape=[
            jax.ShapeDtypeStruct((N, 1), jnp.float32),
            jax.ShapeDtypeStruct((N, D_HID), jnp.float32),
        ],
    )(d0, d1, x, w0)


def _layer1_hs2_body(p0_ref, p1_ref, hs1_ref, inv_ref, b0_ref, w1_ref, hs2_ref):
    agg = p0_ref[...] + p1_ref[...] + hs1_ref[...]
    out1 = jnp.maximum(agg * inv_ref[...] + b0_ref[...], 0.0)
    hs2_ref[...] = jnp.dot(out1, w1_ref[...],
                           preferred_element_type=jnp.float32) * inv_ref[...]


def _layer1_hs2(p0, p1, hs1, inv, b0r, w1):
    return pl.pallas_call(
        _layer1_hs2_body,
        grid=(_GRID,),
        in_specs=[
            pl.BlockSpec((_GRID_R, D_HID), lambda i: (i, 0)),
            pl.BlockSpec((_GRID_R, D_HID), lambda i: (i, 0)),
            pl.BlockSpec((_GRID_R, D_HID), lambda i: (i, 0)),
            pl.BlockSpec((_GRID_R, 1), lambda i: (i, 0)),
            pl.BlockSpec((1, D_HID), lambda i: (0, 0)),
            pl.BlockSpec((D_HID, N_CLS), lambda i: (0, 0)),
        ],
        out_specs=pl.BlockSpec((_GRID_R, N_CLS), lambda i: (i, 0)),
        out_shape=jax.ShapeDtypeStruct((N, N_CLS), jnp.float32),
    )(p0, p1, hs1, inv, b0r, w1)


def _layer2_out_body(p0_ref, p1_ref, hs2_ref, inv_ref, b1_ref, out_ref):
    agg = p0_ref[...] + p1_ref[...] + hs2_ref[...]
    out_ref[...] = agg * inv_ref[...] + b1_ref[...]


def _layer2_out(p0, p1, hs2, inv, b1r):
    return pl.pallas_call(
        _layer2_out_body,
        grid=(_GRID,),
        in_specs=[
            pl.BlockSpec((_GRID_R, N_CLS), lambda i: (i, 0)),
            pl.BlockSpec((_GRID_R, N_CLS), lambda i: (i, 0)),
            pl.BlockSpec((_GRID_R, N_CLS), lambda i: (i, 0)),
            pl.BlockSpec((_GRID_R, 1), lambda i: (i, 0)),
            pl.BlockSpec((1, N_CLS), lambda i: (0, 0)),
        ],
        out_specs=pl.BlockSpec((_GRID_R, N_CLS), lambda i: (i, 0)),
        out_shape=jax.ShapeDtypeStruct((N, N_CLS), jnp.float32),
    )(p0, p1, hs2, inv, b1r)


def kernel(x, edge_index, W0, b0, W1, b1):
    src = edge_index[0].astype(jnp.int32)
    dst = edge_index[1].astype(jnp.int32)
    pad_src = jnp.zeros((TOTAL_CH_PAD * CH - E,), dtype=jnp.int32)
    pad_dst = jnp.full((TOTAL_CH_PAD * CH - E,), PAD_IDX, dtype=jnp.int32)
    src_t = jnp.concatenate([src, pad_src]).reshape(TOTAL_CH_PAD, CH)
    dst_t = jnp.concatenate([dst, pad_dst]).reshape(TOTAL_CH_PAD, CH)

    z1 = jnp.zeros((NPAD,), jnp.float32)
    z16 = jnp.zeros((NPAD, D_HID), jnp.float32)
    z40 = jnp.zeros((NPAD, N_CLS), jnp.float32)
    ones = jnp.ones((CH,), jnp.float32)

    degp = _deg_kernel(dst_t, z1, ones)
    d0 = degp[0].reshape(NPAD, 1)
    d1 = degp[1].reshape(NPAD, 1)
    inv, hs1 = _inv_hs1(d0, d1, x, W0)

    p1 = _agg16(src_t, dst_t, hs1, z16)
    hs2 = _layer1_hs2(p1[0], p1[1], hs1, inv, b0.reshape(1, D_HID), W1)

    p2 = _agg40(src_t, dst_t, hs2, z40)
    return _layer2_out(p2[0], p2[1], hs2, inv, b1.reshape(1, N_CLS))


# speedup vs baseline: 54.2942x; 1.2589x over previous
import functools

import jax
import jax.numpy as jnp
from jax import lax
from jax.experimental import pallas as pl
from jax.experimental.pallas import tpu as pltpu
from jax.experimental.pallas import tpu_sc as plsc

N = 10000
E = 320000
D_IN = 128
D_HID = 16
N_CLS = 40

NC = 2
NS = 16
NW = NC * NS
CH = 128
NCH = E // CH
KT = NCH // NW
KREM = NCH - KT * NW
NPAD = 10240
ROWS = NPAD // NS
NB = 8
NGRP = KT // NB
NTAIL = KT - NGRP * NB

_MESH = plsc.VectorSubcoreMesh(core_axis_name="c", subcore_axis_name="s")
_SC_PARAMS = pltpu.CompilerParams(use_tc_tiling_on_sc=False)


@functools.partial(
    pl.kernel,
    out_type=jax.ShapeDtypeStruct((NC, NPAD), jnp.float32),
    mesh=_MESH,
    compiler_params=_SC_PARAMS,
    scratch_types=[
        pltpu.VMEM((KT, CH), jnp.int32),
        pltpu.VMEM((1, CH), jnp.int32),
        pltpu.VMEM((CH,), jnp.float32),
        pltpu.VMEM_SHARED((NPAD,), jnp.float32),
    ] + [pltpu.SemaphoreType.DMA] * NB,
)
def _deg_kernel(edge_hbm, zeros_hbm, ones_hbm, out_hbm,
                idx_v, xidx_v, ones_v, deg_sh, *sems):
    cid = lax.axis_index("c")
    sid = lax.axis_index("s")
    g = sid * NC + cid
    pltpu.sync_copy(zeros_hbm.at[pl.ds(sid * ROWS, ROWS)],
                    deg_sh.at[pl.ds(sid * ROWS, ROWS)])
    pltpu.sync_copy(ones_hbm, ones_v)
    pltpu.sync_copy(edge_hbm.at[1, pl.ds(g * KT, KT)], idx_v)

    @pl.when(g < KREM)
    def _():
        pltpu.sync_copy(edge_hbm.at[1, pl.ds(NW * KT + g, 1)], xidx_v)

    plsc.subcore_barrier()

    def body(gi, carry):
        base = gi * NB
        cps = [pltpu.async_copy(ones_v, deg_sh.at[idx_v.at[base + b]],
                                sems[b], add=True)
               for b in range(NB)]
        for cp in cps:
            cp.wait()
        return carry

    lax.fori_loop(0, NGRP, body, 0)
    tcps = [pltpu.async_copy(ones_v, deg_sh.at[idx_v.at[NGRP * NB + b]],
                             sems[b], add=True)
            for b in range(NTAIL)]
    for cp in tcps:
        cp.wait()

    @pl.when(g < KREM)
    def _():
        pltpu.async_copy(ones_v, deg_sh.at[xidx_v.at[0]],
                         sems[0], add=True).wait()

    plsc.subcore_barrier()
    pltpu.sync_copy(deg_sh.at[pl.ds(sid * ROWS, ROWS)],
                    out_hbm.at[cid, pl.ds(sid * ROWS, ROWS)])


def _make_agg_kernel(D):
    @functools.partial(
        pl.kernel,
        out_type=jax.ShapeDtypeStruct((NC, NPAD, D), jnp.float32),
        mesh=_MESH,
        compiler_params=_SC_PARAMS,
        scratch_types=[
            pltpu.VMEM((KT, CH), jnp.int32),
            pltpu.VMEM((KT, CH), jnp.int32),
            pltpu.VMEM((1, CH), jnp.int32),
            pltpu.VMEM((1, CH), jnp.int32),
            pltpu.VMEM_SHARED((NPAD, D), jnp.float32),
            pltpu.VMEM_SHARED((N, D), jnp.float32),
        ] + [pltpu.VMEM((CH, D), jnp.float32)] * NB
          + [pltpu.SemaphoreType.DMA] * (2 * NB),
    )
    def agg(edge_hbm, hs_hbm, zeros_hbm, out_hbm,
            src_v, dst_v, xsrc_v, xdst_v, agg_sh, hs_sh, *bufs_sems):
        rb = bufs_sems[:NB]
        gsem = bufs_sems[NB:2 * NB]
        ssem = bufs_sems[2 * NB:]
        cid = lax.axis_index("c")
        sid = lax.axis_index("s")
        g = sid * NC + cid
        pltpu.sync_copy(zeros_hbm.at[pl.ds(sid * ROWS, ROWS)],
                        agg_sh.at[pl.ds(sid * ROWS, ROWS)])
        pltpu.sync_copy(edge_hbm.at[0, pl.ds(g * KT, KT)], src_v)
        pltpu.sync_copy(edge_hbm.at[1, pl.ds(g * KT, KT)], dst_v)

        @pl.when(g < KREM)
        def _():
            pltpu.sync_copy(edge_hbm.at[0, pl.ds(NW * KT + g, 1)], xsrc_v)
            pltpu.sync_copy(edge_hbm.at[1, pl.ds(NW * KT + g, 1)], xdst_v)

        pltpu.sync_copy(hs_hbm.at[pl.ds(sid * (N // NS), N // NS)],
                        hs_sh.at[pl.ds(sid * (N // NS), N // NS)])
        plsc.subcore_barrier()

        def pair(b, src_row, dst_row):
            cp = pltpu.async_copy(hs_sh.at[src_row], rb[b], gsem[b])
            return cp, dst_row

        def body(gi, carry):
            base = gi * NB
            gcps = [pltpu.async_copy(hs_sh.at[src_v.at[base + b]],
                                     rb[b], gsem[b])
                    for b in range(NB)]
            scps = []
            for b in range(NB):
                gcps[b].wait()
                scps.append(pltpu.async_copy(
                    rb[b], agg_sh.at[dst_v.at[base + b]], ssem[b], add=True))
            for cp in scps:
                cp.wait()
            return carry

        lax.fori_loop(0, NGRP, body, 0)

        base = NGRP * NB
        gcps = [pltpu.async_copy(hs_sh.at[src_v.at[base + b]], rb[b], gsem[b])
                for b in range(NTAIL)]
        scps = []
        for b in range(NTAIL):
            gcps[b].wait()
            scps.append(pltpu.async_copy(
                rb[b], agg_sh.at[dst_v.at[base + b]], ssem[b], add=True))
        for cp in scps:
            cp.wait()

        @pl.when(g < KREM)
        def _():
            pltpu.async_copy(hs_sh.at[xsrc_v.at[0]], rb[0], gsem[0]).wait()
            pltpu.async_copy(rb[0], agg_sh.at[xdst_v.at[0]],
                             ssem[0], add=True).wait()

        plsc.subcore_barrier()
        pltpu.sync_copy(agg_sh.at[pl.ds(sid * ROWS, ROWS)],
                        out_hbm.at[cid, pl.ds(sid * ROWS, ROWS)])

    return agg


_agg16 = _make_agg_kernel(D_HID)
_agg40 = _make_agg_kernel(N_CLS)


_GRID_R = 2000
_GRID = N // _GRID_R


def _inv_hs1_body(d_ref, x_ref, w_ref, inv_ref, hs_ref):
    deg = d_ref[0] + d_ref[1] + 1.0
    inv = lax.rsqrt(jnp.maximum(deg, 1.0))
    inv_ref[...] = inv
    hs_ref[...] = jnp.dot(x_ref[...], w_ref[...],
                          preferred_element_type=jnp.float32) * inv


def _inv_hs1(degp, x, w0):
    return pl.pallas_call(
        _inv_hs1_body,
        grid=(_GRID,),
        in_specs=[
            pl.BlockSpec((NC, _GRID_R, 1), lambda i: (0, i, 0)),
            pl.BlockSpec((_GRID_R, D_IN), lambda i: (i, 0)),
            pl.BlockSpec((D_IN, D_HID), lambda i: (0, 0)),
        ],
        out_specs=[
            pl.BlockSpec((_GRID_R, 1), lambda i: (i, 0)),
            pl.BlockSpec((_GRID_R, D_HID), lambda i: (i, 0)),
        ],
        out_shape=[
            jax.ShapeDtypeStruct((N, 1), jnp.float32),
            jax.ShapeDtypeStruct((N, D_HID), jnp.float32),
        ],
    )(degp, x, w0)


def _layer1_hs2_body(p_ref, hs1_ref, inv_ref, b0_ref, w1_ref, hs2_ref):
    agg = p_ref[0] + p_ref[1] + hs1_ref[...]
    out1 = jnp.maximum(agg * inv_ref[...] + b0_ref[...], 0.0)
    hs2_ref[...] = jnp.dot(out1, w1_ref[...],
                           preferred_element_type=jnp.float32) * inv_ref[...]


def _layer1_hs2(p1, hs1, inv, b0r, w1):
    return pl.pallas_call(
        _layer1_hs2_body,
        grid=(_GRID,),
        in_specs=[
            pl.BlockSpec((NC, _GRID_R, D_HID), lambda i: (0, i, 0)),
            pl.BlockSpec((_GRID_R, D_HID), lambda i: (i, 0)),
            pl.BlockSpec((_GRID_R, 1), lambda i: (i, 0)),
            pl.BlockSpec((1, D_HID), lambda i: (0, 0)),
            pl.BlockSpec((D_HID, N_CLS), lambda i: (0, 0)),
        ],
        out_specs=pl.BlockSpec((_GRID_R, N_CLS), lambda i: (i, 0)),
        out_shape=jax.ShapeDtypeStruct((N, N_CLS), jnp.float32),
    )(p1, hs1, inv, b0r, w1)


def _layer2_out_body(p_ref, hs2_ref, inv_ref, b1_ref, out_ref):
    agg = p_ref[0] + p_ref[1] + hs2_ref[...]
    out_ref[...] = agg * inv_ref[...] + b1_ref[...]


def _layer2_out(p2, hs2, inv, b1r):
    return pl.pallas_call(
        _layer2_out_body,
        grid=(_GRID,),
        in_specs=[
            pl.BlockSpec((NC, _GRID_R, N_CLS), lambda i: (0, i, 0)),
            pl.BlockSpec((_GRID_R, N_CLS), lambda i: (i, 0)),
            pl.BlockSpec((_GRID_R, 1), lambda i: (i, 0)),
            pl.BlockSpec((1, N_CLS), lambda i: (0, 0)),
        ],
        out_specs=pl.BlockSpec((_GRID_R, N_CLS), lambda i: (i, 0)),
        out_shape=jax.ShapeDtypeStruct((N, N_CLS), jnp.float32),
    )(p2, hs2, inv, b1r)


def kernel(x, edge_index, W0, b0, W1, b1):
    edge_t = edge_index.astype(jnp.int32).reshape(2, NCH, CH)

    z1 = jnp.zeros((NPAD,), jnp.float32)
    z16 = jnp.zeros((NPAD, D_HID), jnp.float32)
    z40 = jnp.zeros((NPAD, N_CLS), jnp.float32)
    ones = jnp.ones((CH,), jnp.float32)

    degp = _deg_kernel(edge_t, z1, ones)
    inv, hs1 = _inv_hs1(degp.reshape(NC, NPAD, 1), x, W0)

    p1 = _agg16(edge_t, hs1, z16)
    hs2 = _layer1_hs2(p1, hs1, inv, b0.reshape(1, D_HID), W1)

    p2 = _agg40(edge_t, hs2, z40)
    return _layer2_out(p2, hs2, inv, b1.reshape(1, N_CLS))
